# baseline (speedup 1.0000x reference)
import functools

import jax
import jax.numpy as jnp
from jax import lax
from jax.experimental import pallas as pl
from jax.experimental.pallas import tpu as pltpu
from jax.experimental.pallas import tpu_sc as plsc


def _dist_argmin_body(f_ref, e_ref, f2_ref, e2_ref, idx_ref, acc_ref):
    n = e_ref.shape[0]
    dot = lax.dot_general(2.0 * f_ref[...], e_ref[...],
                          (((1,), (1,)), ((), ())),
                          preferred_element_type=jnp.float32)
    dist = (f2_ref[...] - dot) + e2_ref[...]
    m = jnp.min(dist, axis=1, keepdims=True)
    iota = lax.broadcasted_iota(jnp.int32, dist.shape, 1)
    idx_ref[...] = jnp.min(jnp.where(dist == m, iota, n), axis=1, keepdims=True)

    @pl.when(pl.program_id(0) == 0)
    def _():
        acc_ref[0, 0] = 0.0

    acc_ref[0, 0] += jnp.sum(m)


def _dist_argmin(flat, embedding, f2, e2, bm, row0, rows):
    d = flat.shape[1]
    n = embedding.shape[0]
    off = row0 // bm
    return pl.pallas_call(
        _dist_argmin_body,
        grid=(rows // bm,),
        in_specs=[
            pl.BlockSpec((bm, d), lambda i: (i + off, 0)),
            pl.BlockSpec((n, d), lambda i: (0, 0)),
            pl.BlockSpec((bm, 1), lambda i: (i + off, 0)),
            pl.BlockSpec((1, n), lambda i: (0, 0)),
        ],
        out_specs=[
            pl.BlockSpec((bm, 1), lambda i: (i, 0)),
            pl.BlockSpec((1, 1), lambda i: (0, 0), memory_space=pltpu.SMEM),
        ],
        out_shape=[
            jax.ShapeDtypeStruct((rows, 1), jnp.int32),
            jax.ShapeDtypeStruct((1, 1), jnp.float32),
        ],
    )(flat, embedding, f2, e2)


def _make_sc_gather(d, b):
    info = plsc.get_sparse_core_info()
    nc, ns = info.num_cores, info.num_subcores
    nw = nc * ns
    assert b % (8 * nw) == 0
    b_per_w = b // nw
    chunks = 1
    while (b_per_w // chunks) * d * 4 > 150_000 or b_per_w % chunks:
        chunks += 1
    rpc = b_per_w // chunks
    mesh = plsc.VectorSubcoreMesh(core_axis_name="c", subcore_axis_name="s")

    @functools.partial(
        pl.kernel,
        out_type=jax.ShapeDtypeStruct((b, d), jnp.float32),
        mesh=mesh,
        scratch_types=[
            pltpu.VMEM((b_per_w,), jnp.int32),
            pltpu.VMEM((rpc, d), jnp.float32),
            pltpu.VMEM((rpc, d), jnp.float32),
            pltpu.SemaphoreType.DMA,
            pltpu.SemaphoreType.DMA,
            pltpu.SemaphoreType.DMA,
            pltpu.SemaphoreType.DMA,
        ],
    )
    def gather(table_hbm, idx_hbm, out_hbm, idx_v, buf0, buf1, g0, g1, s0, s1):
        wid = lax.axis_index("s") * nc + lax.axis_index("c")
        base = wid * b_per_w
        bufs, gsem, ssem = (buf0, buf1), (g0, g1), (s0, s1)
        pltpu.sync_copy(idx_hbm.at[pl.ds(base, b_per_w)], idx_v)

        gathers = [None, None]
        scatters = [None, None]
        gathers[0] = pltpu.async_copy(
            table_hbm.at[idx_v.at[pl.ds(0, rpc)]], bufs[0], gsem[0])
        for c in range(chunks):
            cur = c % 2
            nxt = (c + 1) % 2
            gathers[cur].wait()
            if c + 1 < chunks:
                if scatters[nxt] is not None:
                    scatters[nxt].wait()
                    scatters[nxt] = None
                gathers[nxt] = pltpu.async_copy(
                    table_hbm.at[idx_v.at[pl.ds((c + 1) * rpc, rpc)]],
                    bufs[nxt], gsem[nxt])
            scatters[cur] = pltpu.async_copy(
                bufs[cur], out_hbm.at[pl.ds(base + c * rpc, rpc)], ssem[cur])
        for s in scatters:
            if s is not None:
                s.wait()

    return gather


def kernel(z_e, embedding):
    d = embedding.shape[1]
    n = embedding.shape[0]
    flat = z_e.reshape(-1, d)
    m = flat.shape[0]
    f2 = jnp.sum(flat ** 2, axis=1, keepdims=True)
    e2 = jnp.sum(embedding ** 2, axis=1).reshape(1, n)

    stages = 2
    ms = m // stages
    sc_gather = _make_sc_gather(d, ms)
    idx_parts, zq_parts, loss_parts = [], [], []
    for s in range(stages):
        idx2, loss_sum = _dist_argmin(flat, embedding, f2, e2, bm=1152,
                                      row0=s * ms, rows=ms)
        idx_parts.append(idx2.reshape(ms))
        loss_parts.append(loss_sum[0, 0])
        zq_parts.append(sc_gather(embedding, idx_parts[-1]))

    idx = jnp.concatenate(idx_parts)
    z_q = jnp.concatenate(zq_parts)
    vq_loss = sum(loss_parts) * (1.25 / (m * d))
    return z_q.reshape(z_e.shape), idx, vq_loss


# speedup vs baseline: 1.1661x; 1.1661x over previous
import functools

import jax
import jax.numpy as jnp
from jax import lax
from jax.experimental import pallas as pl
from jax.experimental.pallas import tpu as pltpu
from jax.experimental.pallas import tpu_sc as plsc


def _dist_argmin_body(f_ref, e_ref, f2_ref, e2_ref, idx_ref, acc_ref):
    n = e_ref.shape[0]
    dot = lax.dot_general(2.0 * f_ref[...], e_ref[...],
                          (((1,), (1,)), ((), ())),
                          preferred_element_type=jnp.float32)
    dist = (lax.transpose(f2_ref[...], (1, 0)) - dot) + e2_ref[...]
    m = jnp.min(dist, axis=1, keepdims=True)
    iota = lax.broadcasted_iota(jnp.int32, dist.shape, 1)
    idx = jnp.min(jnp.where(dist == m, iota, n), axis=1, keepdims=True)
    idx_ref[...] = lax.transpose(idx, (1, 0))

    @pl.when(pl.program_id(0) == 0)
    def _():
        acc_ref[0, 0] = 0.0

    acc_ref[0, 0] += jnp.sum(m)


def _dist_argmin(flat, embedding, f2r, e2, bm, row0, rows):
    d = flat.shape[1]
    n = embedding.shape[0]
    off = row0 // bm
    return pl.pallas_call(
        _dist_argmin_body,
        grid=(rows // bm,),
        in_specs=[
            pl.BlockSpec((bm, d), lambda i: (i + off, 0)),
            pl.BlockSpec((n, d), lambda i: (0, 0)),
            pl.BlockSpec((1, bm), lambda i: (0, i + off)),
            pl.BlockSpec((1, n), lambda i: (0, 0)),
        ],
        out_specs=[
            pl.BlockSpec((1, bm), lambda i: (0, i)),
            pl.BlockSpec((1, 1), lambda i: (0, 0), memory_space=pltpu.SMEM),
        ],
        out_shape=[
            jax.ShapeDtypeStruct((1, rows), jnp.int32),
            jax.ShapeDtypeStruct((1, 1), jnp.float32),
        ],
    )(flat, embedding, f2r, e2)


def _make_sc_gather(d, b, out_rows):
    info = plsc.get_sparse_core_info()
    nc, ns = info.num_cores, info.num_subcores
    nw = nc * ns
    assert b % (8 * nw) == 0
    b_per_w = b // nw
    chunks = 1
    while (b_per_w // chunks) * d * 4 > 230_000 or b_per_w % chunks:
        chunks += 1
    rpc = b_per_w // chunks
    mesh = plsc.VectorSubcoreMesh(core_axis_name="c", subcore_axis_name="s")

    @functools.partial(
        pl.kernel,
        out_type=jax.ShapeDtypeStruct((out_rows, d), jnp.float32),
        mesh=mesh,
        scratch_types=[
            pltpu.VMEM((b_per_w,), jnp.int32),
            pltpu.VMEM((rpc, d), jnp.float32),
            pltpu.VMEM((rpc, d), jnp.float32),
            pltpu.SemaphoreType.DMA,
            pltpu.SemaphoreType.DMA,
            pltpu.SemaphoreType.DMA,
            pltpu.SemaphoreType.DMA,
        ],
    )
    def gather(table_hbm, idx_hbm, out_hbm, idx_v, buf0, buf1, g0, g1, s0, s1):
        wid = lax.axis_index("s") * nc + lax.axis_index("c")
        base = wid * b_per_w
        bufs, gsem, ssem = (buf0, buf1), (g0, g1), (s0, s1)
        pltpu.sync_copy(idx_hbm.at[pl.ds(base, b_per_w)], idx_v)

        gathers = [None, None]
        scatters = [None, None]
        gathers[0] = pltpu.async_copy(
            table_hbm.at[idx_v.at[pl.ds(0, rpc)]], bufs[0], gsem[0])
        for c in range(chunks):
            cur = c % 2
            nxt = (c + 1) % 2
            gathers[cur].wait()
            if c + 1 < chunks:
                if scatters[nxt] is not None:
                    scatters[nxt].wait()
                    scatters[nxt] = None
                gathers[nxt] = pltpu.async_copy(
                    table_hbm.at[idx_v.at[pl.ds((c + 1) * rpc, rpc)]],
                    bufs[nxt], gsem[nxt])
            scatters[cur] = pltpu.async_copy(
                bufs[cur], out_hbm.at[pl.ds(base + c * rpc, rpc)], ssem[cur])
        for s in scatters:
            if s is not None:
                s.wait()

    return gather


def kernel(z_e, embedding):
    d = embedding.shape[1]
    n = embedding.shape[0]
    flat = z_e.reshape(-1, d)
    m = flat.shape[0]
    f2r = jnp.sum(flat ** 2, axis=1).reshape(1, m)
    e2 = jnp.sum(embedding ** 2, axis=1).reshape(1, n)

    stages = 2
    ms = m // stages
    idx_parts, loss_parts = [], []
    for s in range(stages):
        idxr, loss_sum = _dist_argmin(flat, embedding, f2r, e2, bm=1152,
                                      row0=s * ms, rows=ms)
        idx_parts.append(idxr.reshape(ms))
        loss_parts.append(loss_sum[0, 0])

    zq0 = _make_sc_gather(d, ms, m)(embedding, idx_parts[0])
    zq1 = _make_sc_gather(d, ms, ms)(embedding, idx_parts[1])
    z_q = lax.dynamic_update_slice(zq0, zq1, (ms, 0))

    idx = jnp.concatenate(idx_parts)
    vq_loss = (loss_parts[0] + loss_parts[1]) * (1.25 / (m * d))
    return z_q.reshape(z_e.shape), idx, vq_loss


# speedup vs baseline: 1.1872x; 1.0181x over previous
import functools

import jax
import jax.numpy as jnp
from jax import lax
from jax.experimental import pallas as pl
from jax.experimental.pallas import tpu as pltpu
from jax.experimental.pallas import tpu_sc as plsc


def _dist_argmin_body(f_ref, e_ref, f2_ref, e2_ref, idx_ref, acc_ref):
    n = e_ref.shape[0]
    dot = lax.dot_general(2.0 * f_ref[...], e_ref[...],
                          (((1,), (1,)), ((), ())),
                          preferred_element_type=jnp.float32)
    dist = (lax.transpose(f2_ref[...], (1, 0)) - dot) + e2_ref[...]
    m = jnp.min(dist, axis=1, keepdims=True)
    iota = lax.broadcasted_iota(jnp.int32, dist.shape, 1)
    idx = jnp.min(jnp.where(dist == m, iota, n), axis=1, keepdims=True)
    idx_ref[...] = lax.transpose(idx, (1, 0))

    @pl.when(pl.program_id(0) == 0)
    def _():
        acc_ref[0, 0] = 0.0

    acc_ref[0, 0] += jnp.sum(m)


def _dist_argmin(flat, embedding, f2r, e2, bm, row0, rows):
    d = flat.shape[1]
    n = embedding.shape[0]
    off = row0 // bm
    return pl.pallas_call(
        _dist_argmin_body,
        grid=(rows // bm,),
        in_specs=[
            pl.BlockSpec((bm, d), lambda i: (i + off, 0)),
            pl.BlockSpec((n, d), lambda i: (0, 0)),
            pl.BlockSpec((1, bm), lambda i: (0, i + off)),
            pl.BlockSpec((1, n), lambda i: (0, 0)),
        ],
        out_specs=[
            pl.BlockSpec((1, bm), lambda i: (0, i)),
            pl.BlockSpec((1, 1), lambda i: (0, 0), memory_space=pltpu.SMEM),
        ],
        out_shape=[
            jax.ShapeDtypeStruct((1, rows), jnp.int32),
            jax.ShapeDtypeStruct((1, 1), jnp.float32),
        ],
    )(flat, embedding, f2r, e2)


def _make_sc_gather(d, b, out_rows):
    info = plsc.get_sparse_core_info()
    nc, ns = info.num_cores, info.num_subcores
    nw = nc * ns
    assert b % (8 * nw) == 0
    b_per_w = b // nw
    chunks = 2
    while (b_per_w // chunks) * d * 4 > 230_000 or b_per_w % chunks:
        chunks += 1
    rpc = b_per_w // chunks
    mesh = plsc.VectorSubcoreMesh(core_axis_name="c", subcore_axis_name="s")

    @functools.partial(
        pl.kernel,
        out_type=jax.ShapeDtypeStruct((out_rows, d), jnp.float32),
        mesh=mesh,
        scratch_types=[
            pltpu.VMEM((b_per_w,), jnp.int32),
            pltpu.VMEM((rpc, d), jnp.float32),
            pltpu.VMEM((rpc, d), jnp.float32),
            pltpu.SemaphoreType.DMA,
            pltpu.SemaphoreType.DMA,
            pltpu.SemaphoreType.DMA,
            pltpu.SemaphoreType.DMA,
        ],
    )
    def gather(table_hbm, idx_hbm, out_hbm, idx_v, buf0, buf1, g0, g1, s0, s1):
        wid = lax.axis_index("s") * nc + lax.axis_index("c")
        base = wid * b_per_w
        bufs, gsem, ssem = (buf0, buf1), (g0, g1), (s0, s1)
        pltpu.sync_copy(idx_hbm.at[pl.ds(base, b_per_w)], idx_v)

        gathers = [None, None]
        scatters = [None, None]
        gathers[0] = pltpu.async_copy(
            table_hbm.at[idx_v.at[pl.ds(0, rpc)]], bufs[0], gsem[0])
        for c in range(chunks):
            cur = c % 2
            nxt = (c + 1) % 2
            gathers[cur].wait()
            if c + 1 < chunks:
                if scatters[nxt] is not None:
                    scatters[nxt].wait()
                    scatters[nxt] = None
                gathers[nxt] = pltpu.async_copy(
                    table_hbm.at[idx_v.at[pl.ds((c + 1) * rpc, rpc)]],
                    bufs[nxt], gsem[nxt])
            scatters[cur] = pltpu.async_copy(
                bufs[cur], out_hbm.at[pl.ds(base + c * rpc, rpc)], ssem[cur])
        for s in scatters:
            if s is not None:
                s.wait()

    return gather


def kernel(z_e, embedding):
    d = embedding.shape[1]
    n = embedding.shape[0]
    flat = z_e.reshape(-1, d)
    m = flat.shape[0]
    f2r = jnp.sum(flat ** 2, axis=1).reshape(1, m)
    e2 = jnp.sum(embedding ** 2, axis=1).reshape(1, n)

    m0 = (m * 2) // 3
    m1 = m - m0
    idx0, loss0 = _dist_argmin(flat, embedding, f2r, e2, bm=1024,
                               row0=0, rows=m0)
    idx1, loss1 = _dist_argmin(flat, embedding, f2r, e2, bm=1024,
                               row0=m0, rows=m1)

    zq0 = _make_sc_gather(d, m0, m)(embedding, idx0.reshape(m0))
    zq1 = _make_sc_gather(d, m1, m1)(embedding, idx1.reshape(m1))
    z_q = lax.dynamic_update_slice(zq0, zq1, (m0, 0))

    idx = jnp.concatenate([idx0.reshape(m0), idx1.reshape(m1)])
    vq_loss = (loss0[0, 0] + loss1[0, 0]) * (1.25 / (m * d))
    return z_q.reshape(z_e.shape), idx, vq_loss
